# baseline (speedup 1.0000x reference)
import functools

import jax
import jax.numpy as jnp
from jax.experimental import pallas as pl
from jax.experimental.pallas import tpu as pltpu

_NUM_T = 1000
_PAD_T = 1024
_BATCH = 1024
_FLAT = 3 * 64 * 64
_BROW = 32


def _schedule_table():
    scale = 1000.0 / _NUM_T
    beta = jnp.linspace(scale * 0.0001, scale * 0.02, _NUM_T, dtype=jnp.float32)
    ac = jnp.cumprod(1.0 - beta, axis=0)
    tab = jnp.stack([jnp.sqrt(ac), jnp.sqrt(1.0 - ac)], axis=0)
    return jnp.pad(tab, ((0, 0), (0, _PAD_T - _NUM_T)))


def _body(ts_ref, tab_ref, x0_ref, eps_ref, out_ref):
    ts = ts_ref[0, 0, :]
    onehot = (ts[:, None] == jax.lax.broadcasted_iota(
        jnp.int32, (_BROW, _PAD_T), 1)).astype(jnp.float32)
    c1 = jnp.sum(onehot * tab_ref[0:1, :], axis=1, keepdims=True)
    c2 = jnp.sum(onehot * tab_ref[1:2, :], axis=1, keepdims=True)
    out_ref[...] = c1 * x0_ref[...] + c2 * eps_ref[...]


def kernel(x0, timesteps, eps):
    tab = _schedule_table()
    nb = _BATCH // _BROW
    ts3 = timesteps.astype(jnp.int32).reshape(nb, 1, _BROW)
    x2 = x0.reshape(_BATCH, _FLAT)
    e2 = eps.reshape(_BATCH, _FLAT)
    sample = pl.pallas_call(
        _body,
        grid=(nb,),
        in_specs=[
            pl.BlockSpec((1, 1, _BROW), lambda i: (i, 0, 0)),
            pl.BlockSpec((2, _PAD_T), lambda i: (0, 0)),
            pl.BlockSpec((_BROW, _FLAT), lambda i: (i, 0)),
            pl.BlockSpec((_BROW, _FLAT), lambda i: (i, 0)),
        ],
        out_specs=pl.BlockSpec((_BROW, _FLAT), lambda i: (i, 0)),
        out_shape=jax.ShapeDtypeStruct((_BATCH, _FLAT), jnp.float32),
        compiler_params=pltpu.CompilerParams(
            dimension_semantics=("arbitrary",)),
    )(ts3, tab, x2, e2)
    return (sample.reshape(x0.shape), eps)
